# baseline (speedup 1.0000x reference)
import functools

import jax
import jax.numpy as jnp
from jax import lax
from jax.experimental import pallas as pl
from jax.experimental.pallas import tpu as pltpu
from jax.experimental.pallas import tpu_sc as plsc

_NC = 2
_NS = 16
_NW = _NC * _NS
_D = 1024
_ROWS = 256
_RPW = _ROWS // _NW

_mesh = plsc.VectorSubcoreMesh(core_axis_name="c", subcore_axis_name="s")


@functools.partial(
    pl.kernel,
    out_type=jax.ShapeDtypeStruct((_ROWS, _D), jnp.float32),
    mesh=_mesh,
    scratch_types=[
        pltpu.VMEM((_RPW,), jnp.int32),
        pltpu.VMEM((_RPW, _D), jnp.float32),
        pltpu.SemaphoreType.DMA,
    ],
)
def _gather_rows(table_hbm, idx_hbm, out_hbm, idx_v, rows_v, sem):
    wid = lax.axis_index("s") * _NC + lax.axis_index("c")
    base = wid * _RPW
    pltpu.sync_copy(idx_hbm.at[pl.ds(base, _RPW)], idx_v)
    pltpu.async_copy(table_hbm.at[idx_v], rows_v, sem).wait()
    pltpu.sync_copy(rows_v, out_hbm.at[pl.ds(base, _RPW)])


def kernel(para, batch_idx):
    n, b, u = para.shape
    table = para.reshape(n * _ROWS, _D)
    idx = jnp.int32(batch_idx) * _ROWS + lax.iota(jnp.int32, _ROWS)
    out = _gather_rows(table, idx)
    return out.reshape(b, u, 1)


# speedup vs baseline: 2.3112x; 2.3112x over previous
import functools

import jax
import jax.numpy as jnp
from jax import lax
from jax.experimental import pallas as pl
from jax.experimental.pallas import tpu as pltpu
from jax.experimental.pallas import tpu_sc as plsc

_NC = 2
_NS = 16
_NW = _NC * _NS
_SLAB = 4096
_RPW = _SLAB // _NW
_U = 64

_mesh = plsc.VectorSubcoreMesh(core_axis_name="c", subcore_axis_name="s")


@functools.partial(
    pl.kernel,
    out_type=jax.ShapeDtypeStruct((_SLAB, _U), jnp.float32),
    mesh=_mesh,
    scratch_types=[
        pltpu.VMEM((16,), jnp.int32),
    ],
    compiler_params=pltpu.CompilerParams(
        use_tc_tiling_on_sc=True, needs_layout_passes=False
    ),
)
def _copy_slab(table_hbm, idx_hbm, out_hbm, idx_v):
    wid = lax.axis_index("s") * _NC + lax.axis_index("c")
    pltpu.sync_copy(idx_hbm, idx_v)
    b = jnp.max(idx_v[...])
    base = b * _SLAB + wid * _RPW
    pltpu.sync_copy(
        table_hbm.at[pl.ds(base, _RPW), :],
        out_hbm.at[pl.ds(wid * _RPW, _RPW), :],
    )


def kernel(para, batch_idx):
    n, b, u = para.shape
    table = para.reshape(n * b, u)
    idx = jnp.full((16,), batch_idx, jnp.int32)
    out = _copy_slab(table, idx)
    return out.reshape(b, u, 1)


# speedup vs baseline: 25.1798x; 10.8946x over previous
import functools

import jax
import jax.numpy as jnp
from jax import lax
from jax.experimental import pallas as pl
from jax.experimental.pallas import tpu as pltpu
from jax.experimental.pallas import tpu_sc as plsc

_NC = 2
_NS = 16
_NW = _NC * _NS
_B = 4096
_U = 64
_UPW = _U // _NW

_mesh = plsc.VectorSubcoreMesh(core_axis_name="c", subcore_axis_name="s")


@functools.partial(
    pl.kernel,
    out_type=jax.ShapeDtypeStruct((_U * _B,), jnp.float32),
    mesh=_mesh,
    scratch_types=[
        pltpu.VMEM((16,), jnp.int32),
        pltpu.VMEM((_UPW * _B,), jnp.float32),
        pltpu.SemaphoreType.DMA,
    ],
    compiler_params=pltpu.CompilerParams(use_tc_tiling_on_sc=True),
)
def _copy_slab(table_hbm, idx_hbm, out_hbm, idx_v, buf_v, sem):
    wid = lax.axis_index("s") * _NC + lax.axis_index("c")
    pltpu.sync_copy(idx_hbm, idx_v)
    b = idx_v[...][0]
    for j in range(_UPW):
        pltpu.async_copy(
            table_hbm.at[b, wid * _UPW + j, :],
            buf_v.at[pl.ds(j * _B, _B)],
            sem,
        )
    for j in range(_UPW):
        pltpu.make_async_copy(
            table_hbm.at[b, wid * _UPW + j, :],
            buf_v.at[pl.ds(j * _B, _B)],
            sem,
        ).wait()
    pltpu.sync_copy(buf_v, out_hbm.at[pl.ds(wid * _UPW * _B, _UPW * _B)])


def kernel(para, batch_idx):
    n, b, u = para.shape
    para_t = jnp.swapaxes(para, 1, 2)
    idx = jnp.full((16,), batch_idx, jnp.int32)
    flat = _copy_slab(para_t, idx)
    return jnp.swapaxes(flat.reshape(u, b), 0, 1)[:, :, None]


# speedup vs baseline: 29.2167x; 1.1603x over previous
import functools

import jax
import jax.numpy as jnp
from jax import lax
from jax.experimental import pallas as pl
from jax.experimental.pallas import tpu as pltpu
from jax.experimental.pallas import tpu_sc as plsc

_NC = 2
_NS = 16
_NW = _NC * _NS
_B = 4096
_U = 64
_UPW = _U // _NW

_mesh = plsc.VectorSubcoreMesh(core_axis_name="c", subcore_axis_name="s")


@functools.partial(
    pl.kernel,
    out_type=jax.ShapeDtypeStruct((_U * _B,), jnp.float32),
    mesh=_mesh,
    scratch_types=[
        pltpu.VMEM((16,), jnp.int32),
        pltpu.VMEM((_UPW * _B,), jnp.float32),
        pltpu.SemaphoreType.DMA,
    ],
    compiler_params=pltpu.CompilerParams(use_tc_tiling_on_sc=True),
)
def _copy_slab(table_hbm, idx_hbm, out_hbm, idx_v, buf_v, sem):
    wid = lax.axis_index("s") * _NC + lax.axis_index("c")
    pltpu.sync_copy(idx_hbm, idx_v)
    b = idx_v[...][0]
    for j in range(_UPW):
        pltpu.async_copy(
            table_hbm.at[b, wid * _UPW + j, :],
            buf_v.at[pl.ds(j * _B, _B)],
            sem,
        )
    for j in range(_UPW):
        pltpu.make_async_copy(
            table_hbm.at[b, wid * _UPW + j, :],
            buf_v.at[pl.ds(j * _B, _B)],
            sem,
        ).wait()
    pltpu.sync_copy(buf_v, out_hbm.at[pl.ds(wid * _UPW * _B, _UPW * _B)])


def kernel(para, batch_idx):
    n, b, u = para.shape
    para_t = jnp.swapaxes(para, 1, 2)
    idx = jnp.full((16,), batch_idx, jnp.int32)
    flat = _copy_slab(para_t, idx)
    return jnp.transpose(flat.reshape(u, b, 1), (1, 0, 2))


# speedup vs baseline: 29.3855x; 1.0058x over previous
import functools

import jax
import jax.numpy as jnp
from jax import lax
from jax.experimental import pallas as pl
from jax.experimental.pallas import tpu as pltpu
from jax.experimental.pallas import tpu_sc as plsc

_NC = 2
_NS = 16
_NW = _NC * _NS
_B = 4096
_U = 64
_UPW = _U // _NW

_mesh = plsc.VectorSubcoreMesh(core_axis_name="c", subcore_axis_name="s")


@functools.partial(
    pl.kernel,
    out_type=jax.ShapeDtypeStruct((_U * _B,), jnp.float32),
    mesh=_mesh,
    scratch_types=[
        pltpu.VMEM((16,), jnp.int32),
        pltpu.VMEM((_UPW * _B,), jnp.float32),
        pltpu.SemaphoreType.DMA,
    ],
    compiler_params=pltpu.CompilerParams(
        use_tc_tiling_on_sc=True, skip_device_barrier=True
    ),
)
def _copy_slab(table_hbm, idx_hbm, out_hbm, idx_v, buf_v, sem):
    wid = lax.axis_index("s") * _NC + lax.axis_index("c")
    pltpu.sync_copy(idx_hbm, idx_v)
    b = idx_v[...][0]
    for j in range(_UPW):
        pltpu.async_copy(
            table_hbm.at[b, wid * _UPW + j, :],
            buf_v.at[pl.ds(j * _B, _B)],
            sem,
        )
    for j in range(_UPW):
        pltpu.make_async_copy(
            table_hbm.at[b, wid * _UPW + j, :],
            buf_v.at[pl.ds(j * _B, _B)],
            sem,
        ).wait()
    pltpu.sync_copy(buf_v, out_hbm.at[pl.ds(wid * _UPW * _B, _UPW * _B)])


def kernel(para, batch_idx):
    n, b, u = para.shape
    para_t = jnp.swapaxes(para, 1, 2)
    idx = jnp.full((16,), batch_idx, jnp.int32)
    flat = _copy_slab(para_t, idx)
    return jnp.transpose(flat.reshape(u, b, 1), (1, 0, 2))
